# Initial kernel scaffold; baseline (speedup 1.0000x reference)
import jax
import jax.numpy as jnp
from jax.experimental import pallas as pl


def kernel(x, W1, b1, W2, b2, W3, b3):
    raise NotImplementedError("write your pallas kernel here")


# speedup vs baseline: 105.0389x; 105.0389x over previous
import functools

import jax
import jax.numpy as jnp
import numpy as np
from jax.experimental import pallas as pl

GRID = 16
NPG = GRID * GRID


def _build_ahat_t() -> np.ndarray:
    edges = []
    for i in range(GRID):
        for j in range(GRID):
            cur = i * GRID + j
            if j < GRID - 1:
                edges.append([cur, cur + 1])
            if i < GRID - 1:
                edges.append([cur, cur + GRID])
            if j < GRID - 1 and i < GRID - 1:
                edges.append([cur, cur + GRID + 1])
            if j > 0 and i < GRID - 1:
                edges.append([cur, cur + GRID - 1])
    e = np.asarray(edges, dtype=np.int64).T
    loops = np.arange(NPG, dtype=np.int64)
    r = np.concatenate([e[0], loops])
    c = np.concatenate([e[1], loops])
    deg = np.zeros((NPG,), dtype=np.float64)
    np.add.at(deg, c, 1.0)
    dis = 1.0 / np.sqrt(deg)
    norm = dis[r] * dis[c]
    a = np.zeros((NPG, NPG), dtype=np.float64)
    np.add.at(a, (c, r), norm)
    return np.ascontiguousarray(a.T.astype(np.float32))


_AHAT_T = _build_ahat_t()


def _gcn_kernel(x_ref, at_ref, w1_ref, b1_ref, w2_ref, b2_ref, w3_ref, b3_ref,
                out_ref, *, g: int):
    at = at_ref[...]

    def layer(h, w_ref, b_ref, relu, cdim):
        z = jax.lax.dot_general(
            w_ref[...], h, (((0,), (cdim,)), ((), ())),
            preferred_element_type=jnp.float32)
        cout = z.shape[0]
        zf = z.reshape(cout * g, NPG)
        pf = jnp.dot(zf, at, preferred_element_type=jnp.float32)
        p = pf.reshape(cout, g, NPG) + b_ref[...][:, :, None]
        return jnp.maximum(p, 0.0) if relu else p

    h = layer(x_ref[...], w1_ref, b1_ref, True, 1)
    h = layer(h, w2_ref, b2_ref, True, 0)
    h = layer(h, w3_ref, b3_ref, False, 0)
    out_ref[...] = jnp.transpose(h, (1, 2, 0))


@functools.partial(jax.jit, static_argnames=())
def kernel(x, W1, b1, W2, b2, W3, b3):
    B, Cin, H, W_ = x.shape
    Cout = W1.shape[1]
    G = 16
    xg = x.reshape(B, Cin, NPG)
    at = jnp.asarray(_AHAT_T)
    b1c = b1.reshape(Cout, 1)
    b2c = b2.reshape(Cout, 1)
    b3c = b3.reshape(Cout, 1)

    h = pl.pallas_call(
        functools.partial(_gcn_kernel, g=G),
        grid=(B // G,),
        in_specs=[
            pl.BlockSpec((G, Cin, NPG), lambda i: (i, 0, 0)),
            pl.BlockSpec((NPG, NPG), lambda i: (0, 0)),
            pl.BlockSpec((Cin, Cout), lambda i: (0, 0)),
            pl.BlockSpec((Cout, 1), lambda i: (0, 0)),
            pl.BlockSpec((Cout, Cout), lambda i: (0, 0)),
            pl.BlockSpec((Cout, 1), lambda i: (0, 0)),
            pl.BlockSpec((Cout, Cout), lambda i: (0, 0)),
            pl.BlockSpec((Cout, 1), lambda i: (0, 0)),
        ],
        out_specs=pl.BlockSpec((G, NPG, Cout), lambda i: (i, 0, 0)),
        out_shape=jax.ShapeDtypeStruct((B, NPG, Cout), jnp.float32),
    )(xg, at, W1, b1c, W2, b2c, W3, b3c)

    return h.reshape(B, NPG * Cout).reshape(B, Cout, GRID, GRID)


# speedup vs baseline: 135.4753x; 1.2898x over previous
import functools

import jax
import jax.numpy as jnp
import numpy as np
from jax.experimental import pallas as pl

GRID = 16
NPG = GRID * GRID


def _build_ahat_t() -> np.ndarray:
    edges = []
    for i in range(GRID):
        for j in range(GRID):
            cur = i * GRID + j
            if j < GRID - 1:
                edges.append([cur, cur + 1])
            if i < GRID - 1:
                edges.append([cur, cur + GRID])
            if j < GRID - 1 and i < GRID - 1:
                edges.append([cur, cur + GRID + 1])
            if j > 0 and i < GRID - 1:
                edges.append([cur, cur + GRID - 1])
    e = np.asarray(edges, dtype=np.int64).T
    loops = np.arange(NPG, dtype=np.int64)
    r = np.concatenate([e[0], loops])
    c = np.concatenate([e[1], loops])
    deg = np.zeros((NPG,), dtype=np.float64)
    np.add.at(deg, c, 1.0)
    dis = 1.0 / np.sqrt(deg)
    norm = dis[r] * dis[c]
    a = np.zeros((NPG, NPG), dtype=np.float64)
    np.add.at(a, (c, r), norm)
    return np.ascontiguousarray(a.T.astype(np.float32))


_AHAT_T = _build_ahat_t()


def _gcn_kernel(x_ref, at_ref, w1_ref, b1_ref, w2_ref, b2_ref, w3_ref, b3_ref,
                out_ref, *, g: int):
    at = at_ref[...]
    w1 = w1_ref[...]
    w2 = w2_ref[...]
    w3 = w3_ref[...]
    b1 = b1_ref[...]
    b2 = b2_ref[...]
    b3 = b3_ref[...]

    def mm(a, b):
        return jnp.dot(a, b, preferred_element_type=jnp.float32)

    for p in range(g // 2):
        xp = x_ref[2 * p:2 * p + 2].reshape(2 * 128, NPG)
        h = jnp.maximum(mm(mm(w1, xp), at) + b1, 0.0)
        h = jnp.maximum(mm(mm(w2, h), at) + b2, 0.0)
        h = mm(mm(w3, h), at) + b3
        t = jnp.transpose(h, (1, 0))
        out_ref[2 * p] = t[:, :64]
        out_ref[2 * p + 1] = t[:, 64:]


@jax.jit
def kernel(x, W1, b1, W2, b2, W3, b3):
    B, Cin, H, W_ = x.shape
    Cout = W1.shape[1]
    G = 16
    xg = x.reshape(B, Cin, NPG)
    at = jnp.asarray(_AHAT_T)

    def blockdiag2(w):
        ci, co = w.shape
        wt = w.T
        z = jnp.zeros((2 * co, 2 * ci), dtype=w.dtype)
        return z.at[:co, :ci].set(wt).at[co:, ci:].set(wt)

    w1bd = blockdiag2(W1)
    w2bd = blockdiag2(W2)
    w3bd = blockdiag2(W3)
    b1bd = jnp.concatenate([b1, b1]).reshape(2 * Cout, 1)
    b2bd = jnp.concatenate([b2, b2]).reshape(2 * Cout, 1)
    b3bd = jnp.concatenate([b3, b3]).reshape(2 * Cout, 1)

    h = pl.pallas_call(
        functools.partial(_gcn_kernel, g=G),
        grid=(B // G,),
        in_specs=[
            pl.BlockSpec((G, Cin, NPG), lambda i: (i, 0, 0)),
            pl.BlockSpec((NPG, NPG), lambda i: (0, 0)),
            pl.BlockSpec((2 * Cout, 2 * Cin), lambda i: (0, 0)),
            pl.BlockSpec((2 * Cout, 1), lambda i: (0, 0)),
            pl.BlockSpec((2 * Cout, 2 * Cout), lambda i: (0, 0)),
            pl.BlockSpec((2 * Cout, 1), lambda i: (0, 0)),
            pl.BlockSpec((2 * Cout, 2 * Cout), lambda i: (0, 0)),
            pl.BlockSpec((2 * Cout, 1), lambda i: (0, 0)),
        ],
        out_specs=pl.BlockSpec((G, NPG, Cout), lambda i: (i, 0, 0)),
        out_shape=jax.ShapeDtypeStruct((B, NPG, Cout), jnp.float32),
    )(xg, at, w1bd, b1bd, w2bd, b2bd, w3bd, b3bd)

    return h.reshape(B, NPG * Cout).reshape(B, Cout, GRID, GRID)


# speedup vs baseline: 137.7004x; 1.0164x over previous
import functools

import jax
import jax.numpy as jnp
import numpy as np
from jax.experimental import pallas as pl

GRID = 16
NPG = GRID * GRID


def _build_ahat_t() -> np.ndarray:
    edges = []
    for i in range(GRID):
        for j in range(GRID):
            cur = i * GRID + j
            if j < GRID - 1:
                edges.append([cur, cur + 1])
            if i < GRID - 1:
                edges.append([cur, cur + GRID])
            if j < GRID - 1 and i < GRID - 1:
                edges.append([cur, cur + GRID + 1])
            if j > 0 and i < GRID - 1:
                edges.append([cur, cur + GRID - 1])
    e = np.asarray(edges, dtype=np.int64).T
    loops = np.arange(NPG, dtype=np.int64)
    r = np.concatenate([e[0], loops])
    c = np.concatenate([e[1], loops])
    deg = np.zeros((NPG,), dtype=np.float64)
    np.add.at(deg, c, 1.0)
    dis = 1.0 / np.sqrt(deg)
    norm = dis[r] * dis[c]
    a = np.zeros((NPG, NPG), dtype=np.float64)
    np.add.at(a, (c, r), norm)
    return np.ascontiguousarray(a.T.astype(np.float32))


_AHAT_T = _build_ahat_t()


def _gcn_kernel(x_ref, at_ref, w1_ref, b1_ref, w2_ref, b2_ref, w3_ref, b3_ref,
                out_ref, *, g: int):
    at = at_ref[...]
    w1 = w1_ref[...]
    w2 = w2_ref[...]
    w3 = w3_ref[...]
    b1 = b1_ref[...]
    b2 = b2_ref[...]
    b3 = b3_ref[...]

    def mm(a, b):
        return jnp.dot(a, b, preferred_element_type=jnp.float32)

    for p in range(g // 2):
        xp = x_ref[2 * p:2 * p + 2].reshape(2 * 128, NPG)
        h = jnp.maximum(mm(mm(w1, xp), at) + b1, 0.0)
        h = jnp.maximum(mm(mm(w2, h), at) + b2, 0.0)
        h = mm(mm(w3, h), at) + b3
        t = jnp.transpose(h, (1, 0))
        out_ref[2 * p] = t[:, :64]
        out_ref[2 * p + 1] = t[:, 64:]


@jax.jit
def kernel(x, W1, b1, W2, b2, W3, b3):
    B, Cin, H, W_ = x.shape
    Cout = W1.shape[1]
    G = 32
    xg = x.reshape(B, Cin, NPG)
    at = jnp.asarray(_AHAT_T)

    def blockdiag2(w):
        ci, co = w.shape
        wt = w.T
        z = jnp.zeros((2 * co, 2 * ci), dtype=w.dtype)
        return z.at[:co, :ci].set(wt).at[co:, ci:].set(wt)

    w1bd = blockdiag2(W1)
    w2bd = blockdiag2(W2)
    w3bd = blockdiag2(W3)
    b1bd = jnp.concatenate([b1, b1]).reshape(2 * Cout, 1)
    b2bd = jnp.concatenate([b2, b2]).reshape(2 * Cout, 1)
    b3bd = jnp.concatenate([b3, b3]).reshape(2 * Cout, 1)

    h = pl.pallas_call(
        functools.partial(_gcn_kernel, g=G),
        grid=(B // G,),
        in_specs=[
            pl.BlockSpec((G, Cin, NPG), lambda i: (i, 0, 0)),
            pl.BlockSpec((NPG, NPG), lambda i: (0, 0)),
            pl.BlockSpec((2 * Cout, 2 * Cin), lambda i: (0, 0)),
            pl.BlockSpec((2 * Cout, 1), lambda i: (0, 0)),
            pl.BlockSpec((2 * Cout, 2 * Cout), lambda i: (0, 0)),
            pl.BlockSpec((2 * Cout, 1), lambda i: (0, 0)),
            pl.BlockSpec((2 * Cout, 2 * Cout), lambda i: (0, 0)),
            pl.BlockSpec((2 * Cout, 1), lambda i: (0, 0)),
        ],
        out_specs=pl.BlockSpec((G, NPG, Cout), lambda i: (i, 0, 0)),
        out_shape=jax.ShapeDtypeStruct((B, NPG, Cout), jnp.float32),
    )(xg, at, w1bd, b1bd, w2bd, b2bd, w3bd, b3bd)

    return h.reshape(B, NPG * Cout).reshape(B, Cout, GRID, GRID)
